# Initial kernel scaffold; baseline (speedup 1.0000x reference)
import functools
import math

import jax
import jax.numpy as jnp
from jax import lax
from jax.experimental import pallas as pl
from jax.experimental.pallas import tpu as pltpu
from jax.experimental.pallas import tpu_sc as plsc

S = 8192
D = 1024
HID = 4096
E = 16
TOPK = 2
CAP = int(math.ceil(1.25 * (S * TOPK) / E))
SLOTS = S * TOPK

_NB = 8
_BS = S // _NB
_HBLK = 1024
_NH = HID // _HBLK

_NC = 2
_NS = 16
_NW = _NC * _NS
_GCH = 64
_NGCH = CAP // _GCH
_TPW = S // _NW
_CCH = 16
_NCCH = _TPW // _CCH


def _cumsum_excl(c):
    inc = c
    k = 1
    while k < S:
        shifted = jnp.concatenate(
            [jnp.zeros((k, E), jnp.int32), inc[: S - k, :]], axis=0)
        inc = inc + shifted
        k *= 2
    return inc - c


def _router_body(x_ref, wg_ref, eid2_ref, posn2_ref, fpos2_ref, gates2_ref,
                 aux_ref, scores_s):
    b = pl.program_id(0)
    xb = x_ref[...]
    wg = wg_ref[...]
    logits = lax.dot_general(xb, wg, (((1,), (1,)), ((), ())),
                             preferred_element_type=jnp.float32)
    m = jnp.max(logits, axis=-1, keepdims=True)
    ex = jnp.exp(logits - m)
    scores_s[pl.ds(b * _BS, _BS), :] = ex / jnp.sum(ex, axis=-1, keepdims=True)

    @pl.when(b == _NB - 1)
    def _route():
        scores = scores_s[...]
        eidx = lax.broadcasted_iota(jnp.int32, (S, E), 1)
        v0 = jnp.max(scores, axis=-1, keepdims=True)
        e0 = jnp.min(jnp.where(scores == v0, eidx, E), axis=-1, keepdims=True)
        sel0 = eidx == e0
        s2 = jnp.where(sel0, -1.0, scores)
        v1 = jnp.max(s2, axis=-1, keepdims=True)
        e1 = jnp.min(jnp.where(s2 == v1, eidx, E), axis=-1, keepdims=True)
        sel1 = eidx == e1
        sel = sel0 | sel1
        bits = lax.bitcast_convert_type(scores, jnp.int32)
        count = jnp.sum(sel.astype(jnp.int32), axis=0, keepdims=True)

        def bs_body(_, lohi):
            lo, hi = lohi
            mid = lo + (hi - lo) // 2
            cge = jnp.sum((sel & (bits >= mid)).astype(jnp.int32),
                          axis=0, keepdims=True)
            take = cge >= CAP
            return jnp.where(take, mid, lo), jnp.where(take, hi, mid)

        lo0 = jnp.zeros((1, E), jnp.int32)
        hi0 = jnp.full((1, E), 0x3F800001, jnp.int32)
        vc, _ = lax.fori_loop(0, 30, bs_body, (lo0, hi0))
        gt = sel & (bits > vc)
        eq = sel & (bits == vc)
        cgt = jnp.sum(gt.astype(jnp.int32), axis=0, keepdims=True)
        rr = CAP - cgt
        over = count > CAP
        tie_excl = _cumsum_excl(eq.astype(jnp.int32))
        kept = sel & ((~over) | gt | (eq & (tie_excl < rr)))
        pos = _cumsum_excl(kept.astype(jnp.int32))

        def pick_i(mat, selm):
            return jnp.sum(jnp.where(selm, mat, 0), axis=-1, keepdims=True)

        k0 = pick_i(kept.astype(jnp.int32), sel0)
        k1 = pick_i(kept.astype(jnp.int32), sel1)
        p0 = pick_i(pos, sel0)
        p1 = pick_i(pos, sel1)
        eid2_ref[...] = jnp.concatenate([e0, e1], axis=1)
        posn2_ref[...] = jnp.concatenate(
            [jnp.where(k0 > 0, p0, -1), jnp.where(k1 > 0, p1, -1)], axis=1)
        fpos2_ref[...] = jnp.concatenate(
            [jnp.where(k0 > 0, e0 * CAP + p0, 0),
             jnp.where(k1 > 0, e1 * CAP + p1, 0)], axis=1)
        gates2_ref[...] = jnp.concatenate(
            [jnp.where(k0 > 0, v0, 0.0), jnp.where(k1 > 0, v1, 0.0)], axis=1)
        importance = jnp.sum(scores, axis=0, keepdims=True)
        aux_ref[0, 0] = jnp.sum(importance * count.astype(jnp.float32)) * (
            E / (S * S))


def _router(x, Wg, interpret=False):
    return pl.pallas_call(
        _router_body,
        grid=(_NB,),
        in_specs=[
            pl.BlockSpec((_BS, D), lambda b: (b, 0)),
            pl.BlockSpec((E, D), lambda b: (0, 0)),
        ],
        out_specs=[
            pl.BlockSpec((S, TOPK), lambda b: (0, 0)),
            pl.BlockSpec((S, TOPK), lambda b: (0, 0)),
            pl.BlockSpec((S, TOPK), lambda b: (0, 0)),
            pl.BlockSpec((S, TOPK), lambda b: (0, 0)),
            pl.BlockSpec((1, 1), lambda b: (0, 0)),
        ],
        out_shape=[
            jax.ShapeDtypeStruct((S, TOPK), jnp.int32),
            jax.ShapeDtypeStruct((S, TOPK), jnp.int32),
            jax.ShapeDtypeStruct((S, TOPK), jnp.int32),
            jax.ShapeDtypeStruct((S, TOPK), jnp.float32),
            jax.ShapeDtypeStruct((1, 1), jnp.float32),
        ],
        scratch_shapes=[pltpu.VMEM((S, E), jnp.float32)],
        interpret=interpret,
    )(x, Wg)


def _gelu(h):
    return 0.5 * h * (1.0 + lax.erf(h * 0.7071067811865476))


def _ffn_body(xe_ref, w1_ref, w2_ref, y_ref):
    hb = pl.program_id(1)
    h = _gelu(jnp.dot(xe_ref[...], w1_ref[0],
                      preferred_element_type=jnp.float32))
    part = jnp.dot(h, w2_ref[0], preferred_element_type=jnp.float32)

    @pl.when(hb == 0)
    def _init():
        y_ref[...] = part

    @pl.when(hb > 0)
    def _acc():
        y_ref[...] = y_ref[...] + part


def _ffn(xe, W1, W2, interpret=False):
    return pl.pallas_call(
        _ffn_body,
        grid=(E, _NH),
        in_specs=[
            pl.BlockSpec((CAP, D), lambda e, h: (e, 0)),
            pl.BlockSpec((1, D, _HBLK), lambda e, h: (e, 0, h)),
            pl.BlockSpec((1, _HBLK, D), lambda e, h: (e, h, 0)),
        ],
        out_specs=pl.BlockSpec((CAP, D), lambda e, h: (e, 0)),
        out_shape=jax.ShapeDtypeStruct((E * CAP, D), jnp.float32),
        compiler_params=pltpu.CompilerParams(
            dimension_semantics=("arbitrary", "arbitrary")),
        interpret=interpret,
    )(xe, W1, W2)


def _dispatch_body(x_hbm, eid_hbm, posn_hbm, xe_hbm,
                   eid_v, posn_v, tok_v, rows_v, sem):
    cid = lax.axis_index("c")
    sid = lax.axis_index("s")
    wid = sid * _NC + cid
    e = wid // 2
    half = wid % 2

    def zero_body(i, _):
        tok_v[pl.ds(i * 16, 16)] = jnp.zeros((16,), jnp.int32)
        return 0

    lax.fori_loop(0, CAP // 16, zero_body, 0)
    pltpu.sync_copy(eid_hbm, eid_v)
    pltpu.sync_copy(posn_hbm, posn_v)

    def scan_body(c, _):
        ev = eid_v[pl.ds(c * 16, 16)]
        pv = posn_v[pl.ds(c * 16, 16)]
        m = (ev == e) & (pv >= 0)
        lane = lax.broadcasted_iota(jnp.int32, (16,), 0)
        toks = (c * 16 + lane) // 2
        pvc = jnp.where(m, pv, 0)
        plsc.store_scatter(tok_v, [pvc], toks, mask=m)
        return 0

    lax.fori_loop(0, SLOTS // 16, scan_body, 0)

    for c in range(half, _NGCH, 2):
        idx = tok_v.at[pl.ds(c * _GCH, _GCH)]
        pltpu.async_copy(x_hbm.at[idx], rows_v, sem).wait()
        pltpu.sync_copy(rows_v, xe_hbm.at[pl.ds(e * CAP + c * _GCH, _GCH)])


def _dispatch(x, eid_slot, posn_slot):
    mesh = plsc.VectorSubcoreMesh(core_axis_name="c", subcore_axis_name="s")
    return pl.kernel(
        _dispatch_body,
        out_type=jax.ShapeDtypeStruct((E * CAP, D), jnp.float32),
        mesh=mesh,
        scratch_types=[
            pltpu.VMEM((SLOTS,), jnp.int32),
            pltpu.VMEM((SLOTS,), jnp.int32),
            pltpu.VMEM((CAP,), jnp.int32),
            pltpu.VMEM((_GCH, D), jnp.float32),
            pltpu.SemaphoreType.DMA,
        ],
    )(x, eid_slot, posn_slot)


def _combine_body(y_hbm, fpos_hbm, gates_hbm, out_hbm,
                  fp_v, g_v, rows_v, out_v, sem):
    cid = lax.axis_index("c")
    sid = lax.axis_index("s")
    wid = sid * _NC + cid
    tok0 = wid * _TPW
    pltpu.sync_copy(fpos_hbm.at[pl.ds(tok0 * 2, _TPW * 2)], fp_v)
    pltpu.sync_copy(gates_hbm.at[pl.ds(tok0 * 2, _TPW * 2)], g_v)

    def chunk_body(c, _):
        idx = fp_v.at[pl.ds(c * (2 * _CCH), 2 * _CCH)]
        pltpu.async_copy(y_hbm.at[idx], rows_v, sem).wait()
        for t in range(_CCH):
            s0 = c * (2 * _CCH) + 2 * t
            g0 = plsc.load_gather(g_v, [jnp.full((16,), s0, jnp.int32)])
            g1 = plsc.load_gather(g_v, [jnp.full((16,), s0 + 1, jnp.int32)])

            def col_body(j, _, g0=g0, g1=g1, t=t):
                out_v[t, pl.ds(j * 16, 16)] = (
                    g0 * rows_v[2 * t, pl.ds(j * 16, 16)]
                    + g1 * rows_v[2 * t + 1, pl.ds(j * 16, 16)])
                return 0

            lax.fori_loop(0, D // 16, col_body, 0)
        pltpu.sync_copy(out_v, out_hbm.at[pl.ds(tok0 + c * _CCH, _CCH)])
        return 0

    lax.fori_loop(0, _NCCH, chunk_body, 0)


def _combine(y, fpos_slot, gates_slot):
    mesh = plsc.VectorSubcoreMesh(core_axis_name="c", subcore_axis_name="s")
    return pl.kernel(
        _combine_body,
        out_type=jax.ShapeDtypeStruct((S, D), jnp.float32),
        mesh=mesh,
        scratch_types=[
            pltpu.VMEM((2 * _TPW,), jnp.int32),
            pltpu.VMEM((2 * _TPW,), jnp.float32),
            pltpu.VMEM((2 * _CCH, D), jnp.float32),
            pltpu.VMEM((_CCH, D), jnp.float32),
            pltpu.SemaphoreType.DMA,
        ],
    )(y, fpos_slot, gates_slot)


def kernel(x, Wg, W1, W2):
    eid2, posn2, fpos2, gates2, aux = _router(x, Wg)
    xe = _dispatch(x, eid2.reshape(-1), posn2.reshape(-1))
    y = _ffn(xe, W1, W2)
    out = _combine(y, fpos2.reshape(-1), gates2.reshape(-1))
    return out, aux.reshape(())


# speedup vs baseline: 3.8873x; 3.8873x over previous
import functools
import math

import jax
import jax.numpy as jnp
from jax import lax
from jax.experimental import pallas as pl
from jax.experimental.pallas import tpu as pltpu
from jax.experimental.pallas import tpu_sc as plsc

S = 8192
D = 1024
HID = 4096
E = 16
TOPK = 2
CAP = int(math.ceil(1.25 * (S * TOPK) / E))
SLOTS = S * TOPK

_NB = 8
_BS = S // _NB
_HBLK = 1024
_NH = HID // _HBLK

_NC = 2
_NS = 16
_NW = _NC * _NS
_GCH = 64
_NGCH = CAP // _GCH
_TPW = S // _NW
_CCH = 16
_NCCH = _TPW // _CCH


def _cumsum_excl(c):
    inc = c
    k = 1
    while k < S:
        shifted = jnp.concatenate(
            [jnp.zeros((E, k), jnp.int32), inc[:, : S - k]], axis=1)
        inc = inc + shifted
        k *= 2
    return inc - c


def _router_body(x_ref, wg_ref, eid2_ref, posn2_ref, fpos2_ref, gates2_ref,
                 aux_ref, scores_s):
    b = pl.program_id(0)
    xb = x_ref[...]
    wg = wg_ref[...]
    logits = lax.dot_general(wg, xb, (((1,), (1,)), ((), ())),
                             preferred_element_type=jnp.float32)
    m = jnp.max(logits, axis=0, keepdims=True)
    ex = jnp.exp(logits - m)
    scores_s[:, pl.ds(b * _BS, _BS)] = ex / jnp.sum(ex, axis=0, keepdims=True)

    @pl.when(b == _NB - 1)
    def _route():
        scores = scores_s[...]
        eidx = lax.broadcasted_iota(jnp.int32, (E, S), 0)
        v0 = jnp.max(scores, axis=0, keepdims=True)
        e0 = jnp.min(jnp.where(scores == v0, eidx, E), axis=0, keepdims=True)
        sel0 = eidx == e0
        s2 = jnp.where(sel0, -1.0, scores)
        v1 = jnp.max(s2, axis=0, keepdims=True)
        e1 = jnp.min(jnp.where(s2 == v1, eidx, E), axis=0, keepdims=True)
        sel1 = eidx == e1
        sel = sel0 | sel1
        bits = lax.bitcast_convert_type(scores, jnp.int32)
        count = jnp.sum(sel.astype(jnp.int32), axis=1, keepdims=True)

        def bs_body(_, lohi):
            lo, hi = lohi
            mid = lo + (hi - lo) // 2
            cge = jnp.sum((sel & (bits >= mid)).astype(jnp.int32),
                          axis=1, keepdims=True)
            take = cge >= CAP
            return jnp.where(take, mid, lo), jnp.where(take, hi, mid)

        lo0 = jnp.zeros((E, 1), jnp.int32)
        hi0 = jnp.full((E, 1), 0x3F800001, jnp.int32)
        vc, _ = lax.fori_loop(0, 30, bs_body, (lo0, hi0))
        gt = sel & (bits > vc)
        eq = sel & (bits == vc)
        cgt = jnp.sum(gt.astype(jnp.int32), axis=1, keepdims=True)
        rr = CAP - cgt
        over = count > CAP
        tie_excl = _cumsum_excl(eq.astype(jnp.int32))
        kept = sel & ((~over) | gt | (eq & (tie_excl < rr)))
        pos = _cumsum_excl(kept.astype(jnp.int32))

        def pick_i(mat, selm):
            return jnp.sum(jnp.where(selm, mat, 0), axis=0, keepdims=True)

        k0 = pick_i(kept.astype(jnp.int32), sel0)
        k1 = pick_i(kept.astype(jnp.int32), sel1)
        p0 = pick_i(pos, sel0)
        p1 = pick_i(pos, sel1)
        eid2_ref[...] = jnp.concatenate([e0, e1], axis=0)
        posn2_ref[...] = jnp.concatenate(
            [jnp.where(k0 > 0, p0, -1), jnp.where(k1 > 0, p1, -1)], axis=0)
        fpos2_ref[...] = jnp.concatenate(
            [jnp.where(k0 > 0, e0 * CAP + p0, 0),
             jnp.where(k1 > 0, e1 * CAP + p1, 0)], axis=0)
        gates2_ref[...] = jnp.concatenate(
            [jnp.where(k0 > 0, v0, 0.0), jnp.where(k1 > 0, v1, 0.0)], axis=0)
        importance = jnp.sum(scores, axis=1, keepdims=True)
        aux_ref[...] = jnp.sum(importance * count.astype(jnp.float32),
                               axis=0, keepdims=True) * (E / (S * S))


def _router(x, Wg, interpret=False):
    return pl.pallas_call(
        _router_body,
        grid=(_NB,),
        in_specs=[
            pl.BlockSpec((_BS, D), lambda b: (b, 0)),
            pl.BlockSpec((E, D), lambda b: (0, 0)),
        ],
        out_specs=[
            pl.BlockSpec((TOPK, S), lambda b: (0, 0)),
            pl.BlockSpec((TOPK, S), lambda b: (0, 0)),
            pl.BlockSpec((TOPK, S), lambda b: (0, 0)),
            pl.BlockSpec((TOPK, S), lambda b: (0, 0)),
            pl.BlockSpec((1, 1), lambda b: (0, 0)),
        ],
        out_shape=[
            jax.ShapeDtypeStruct((TOPK, S), jnp.int32),
            jax.ShapeDtypeStruct((TOPK, S), jnp.int32),
            jax.ShapeDtypeStruct((TOPK, S), jnp.int32),
            jax.ShapeDtypeStruct((TOPK, S), jnp.float32),
            jax.ShapeDtypeStruct((1, 1), jnp.float32),
        ],
        scratch_shapes=[pltpu.VMEM((E, S), jnp.float32)],
        compiler_params=pltpu.CompilerParams(
            vmem_limit_bytes=100 * 1024 * 1024),
        interpret=interpret,
    )(x, Wg)


def _gelu(h):
    return 0.5 * h * (1.0 + lax.erf(h * 0.7071067811865476))


def _ffn_body(xe_ref, w1_ref, w2_ref, y_ref):
    hb = pl.program_id(1)
    h = _gelu(jnp.dot(xe_ref[...], w1_ref[0],
                      preferred_element_type=jnp.float32))
    part = jnp.dot(h, w2_ref[0], preferred_element_type=jnp.float32)

    @pl.when(hb == 0)
    def _init():
        y_ref[...] = part

    @pl.when(hb > 0)
    def _acc():
        y_ref[...] = y_ref[...] + part


def _ffn(xe, W1, W2, interpret=False):
    return pl.pallas_call(
        _ffn_body,
        grid=(E, _NH),
        in_specs=[
            pl.BlockSpec((CAP, D), lambda e, h: (e, 0)),
            pl.BlockSpec((1, D, _HBLK), lambda e, h: (e, 0, h)),
            pl.BlockSpec((1, _HBLK, D), lambda e, h: (e, h, 0)),
        ],
        out_specs=pl.BlockSpec((CAP, D), lambda e, h: (e, 0)),
        out_shape=jax.ShapeDtypeStruct((E * CAP, D), jnp.float32),
        compiler_params=pltpu.CompilerParams(
            dimension_semantics=("arbitrary", "arbitrary")),
        interpret=interpret,
    )(xe, W1, W2)


_SPW = SLOTS // _NW
_SCH = 16
_NSCH = _SPW // _SCH


def _dispatch_body(x_hbm, eid_hbm, posn_hbm, xe_hbm,
                   eid_seg, posn_seg, idx16, dst16, rows_v, sem):
    cid = lax.axis_index("c")
    sid = lax.axis_index("s")
    wid = sid * _NC + cid
    base = wid * _SPW
    pltpu.sync_copy(eid_hbm.at[pl.ds(base, _SPW)], eid_seg)
    pltpu.sync_copy(posn_hbm.at[pl.ds(base, _SPW)], posn_seg)

    def chunk_body(c, _):
        ev = eid_seg[pl.ds(c * _SCH, _SCH)]
        pv = posn_seg[pl.ds(c * _SCH, _SCH)]
        lane = lax.broadcasted_iota(jnp.int32, (16,), 0)
        idx16[...] = lax.bitwise_and(base + c * _SCH + lane, S - 1)
        dst16[...] = jnp.where(pv >= 0, ev * CAP + pv, E * CAP + lane)
        pltpu.async_copy(x_hbm.at[idx16], rows_v, sem).wait()
        pltpu.async_copy(rows_v, xe_hbm.at[dst16], sem).wait()
        return 0

    lax.fori_loop(0, _NSCH, chunk_body, 0)


def _dispatch(x, eid_slot, posn_slot):
    mesh = plsc.VectorSubcoreMesh(core_axis_name="c", subcore_axis_name="s")
    return pl.kernel(
        _dispatch_body,
        out_type=jax.ShapeDtypeStruct((E * CAP + 16, D), jnp.float32),
        mesh=mesh,
        scratch_types=[
            pltpu.VMEM((_SPW,), jnp.int32),
            pltpu.VMEM((_SPW,), jnp.int32),
            pltpu.VMEM((_SCH,), jnp.int32),
            pltpu.VMEM((_SCH,), jnp.int32),
            pltpu.VMEM((_SCH, D), jnp.float32),
            pltpu.SemaphoreType.DMA,
        ],
    )(x, eid_slot, posn_slot)


def _combine_body(y_hbm, fpos_hbm, gatesb_hbm, out_hbm,
                  fp_v, gb_v, rows0_v, rows1_v, out_v, sem):
    cid = lax.axis_index("c")
    sid = lax.axis_index("s")
    wid = sid * _NC + cid
    tok0 = wid * _TPW
    pltpu.sync_copy(fpos_hbm.at[pl.ds(tok0, _TPW)], fp_v.at[pl.ds(0, _TPW)])
    pltpu.sync_copy(fpos_hbm.at[pl.ds(S + tok0, _TPW)],
                    fp_v.at[pl.ds(_TPW, _TPW)])
    pltpu.sync_copy(gatesb_hbm.at[pl.ds(tok0, _TPW)],
                    gb_v.at[pl.ds(0, _TPW)])
    pltpu.sync_copy(gatesb_hbm.at[pl.ds(S + tok0, _TPW)],
                    gb_v.at[pl.ds(_TPW, _TPW)])

    def chunk_body(c, _):
        idx0 = fp_v.at[pl.ds(c * _CCH, _CCH)]
        idx1 = fp_v.at[pl.ds(_TPW + c * _CCH, _CCH)]
        pltpu.async_copy(y_hbm.at[idx0], rows0_v, sem).wait()
        pltpu.async_copy(y_hbm.at[idx1], rows1_v, sem).wait()
        for t in range(_CCH):
            g0 = gb_v[c * _CCH + t, pl.ds(0, 16)]
            g1 = gb_v[_TPW + c * _CCH + t, pl.ds(0, 16)]

            def col_body(j, _, g0=g0, g1=g1, t=t):
                z = jnp.zeros((16,), jnp.float32)
                a0 = jnp.where(g0 > 0, g0 * rows0_v[t, pl.ds(j * 16, 16)], z)
                a1 = jnp.where(g1 > 0, g1 * rows1_v[t, pl.ds(j * 16, 16)], z)
                out_v[t, pl.ds(j * 16, 16)] = a0 + a1
                return 0

            lax.fori_loop(0, D // 16, col_body, 0)
        pltpu.sync_copy(out_v, out_hbm.at[pl.ds(tok0 + c * _CCH, _CCH)])
        return 0

    lax.fori_loop(0, _NCCH, chunk_body, 0)


def _combine(y, fpos_slot, gates_bcast):
    mesh = plsc.VectorSubcoreMesh(core_axis_name="c", subcore_axis_name="s")
    return pl.kernel(
        _combine_body,
        out_type=jax.ShapeDtypeStruct((S, D), jnp.float32),
        mesh=mesh,
        scratch_types=[
            pltpu.VMEM((2 * _TPW,), jnp.int32),
            pltpu.VMEM((2 * _TPW, 16), jnp.float32),
            pltpu.VMEM((_CCH, D), jnp.float32),
            pltpu.VMEM((_CCH, D), jnp.float32),
            pltpu.VMEM((_CCH, D), jnp.float32),
            pltpu.SemaphoreType.DMA,
        ],
    )(y, fpos_slot, gates_bcast)


def kernel(x, Wg, W1, W2):
    eid2, posn2, fpos2, gates2, aux = _router(x, Wg)
    gatesb = jnp.broadcast_to(gates2.reshape(SLOTS, 1), (SLOTS, 16))
    xe = _dispatch(x, eid2.reshape(-1), posn2.reshape(-1))
    y = _ffn(xe, W1, W2)
    out = _combine(y, fpos2.reshape(-1), gatesb)
    return out, aux.reshape(())
